# Initial kernel scaffold; baseline (speedup 1.0000x reference)
import jax
import jax.numpy as jnp
from jax.experimental import pallas as pl


def kernel(x, Wk, bk, Wv, bv, Wq, bq, Wo, bo):
    raise NotImplementedError("write your pallas kernel here")


# speedup vs baseline: 69.6155x; 69.6155x over previous
import functools
import math

import jax
import jax.numpy as jnp
from jax import lax
from jax.experimental import pallas as pl
from jax.experimental.pallas import tpu as pltpu

_DECAY = 0.99
_LR = 0.01
_CHUNK = 256


def _fwd_kernel(x_ref, wk_ref, bk_ref, wv_ref, bv_ref, wq_ref, bq_ref,
                wo_ref, bo_ref, y_ref, state_ref, *, C, ln_decay):
    @pl.when(pl.program_id(1) == 0)
    def _():
        state_ref[...] = jnp.zeros_like(state_ref)

    xc = x_ref[0]
    c11 = (((1,), (1,)), ((), ()))
    k = lax.dot_general(xc, wk_ref[...], c11,
                        preferred_element_type=jnp.float32) + bk_ref[...]
    v = lax.dot_general(xc, wv_ref[...], c11,
                        preferred_element_type=jnp.float32) + bv_ref[...]
    q = lax.dot_general(xc, wq_ref[...], c11,
                        preferred_element_type=jnp.float32) + bq_ref[...]

    state = state_ref[0]

    i_cm = lax.broadcasted_iota(jnp.int32, k.shape, 0).astype(jnp.float32)
    inter = lax.dot_general(q, state, c11, preferred_element_type=jnp.float32)
    inter = inter * jnp.exp(i_cm * ln_decay)

    ii = lax.broadcasted_iota(jnp.int32, (C, C), 0).astype(jnp.float32)
    jj = lax.broadcasted_iota(jnp.int32, (C, C), 1).astype(jnp.float32)
    mask = jnp.where(jj < ii, jnp.exp((ii - 1.0 - jj) * ln_decay), 0.0)
    a = lax.dot_general(q, k, c11, preferred_element_type=jnp.float32) * mask
    intra = lax.dot_general(a, v, (((1,), (0,)), ((), ())),
                            preferred_element_type=jnp.float32)

    outs = inter + _LR * intra
    y_ref[0] = lax.dot_general(outs, wo_ref[...], c11,
                               preferred_element_type=jnp.float32) + bo_ref[...]

    w = jnp.exp((C - 1.0 - i_cm) * ln_decay)
    supd = lax.dot_general(v * w, k, (((0,), (0,)), ((), ())),
                           preferred_element_type=jnp.float32)
    state_ref[0] = (_DECAY ** C) * state + _LR * supd


def kernel(x, Wk, bk, Wv, bv, Wq, bq, Wo, bo):
    B, S, D = x.shape
    M = Wk.shape[0]
    C = _CHUNK
    assert S % C == 0
    body = functools.partial(_fwd_kernel, C=C, ln_decay=math.log(_DECAY))
    y, state = pl.pallas_call(
        body,
        grid=(B, S // C),
        in_specs=[
            pl.BlockSpec((1, C, D), lambda b, c: (b, c, 0)),
            pl.BlockSpec((M, D), lambda b, c: (0, 0)),
            pl.BlockSpec((1, M), lambda b, c: (0, 0)),
            pl.BlockSpec((M, D), lambda b, c: (0, 0)),
            pl.BlockSpec((1, M), lambda b, c: (0, 0)),
            pl.BlockSpec((M, D), lambda b, c: (0, 0)),
            pl.BlockSpec((1, M), lambda b, c: (0, 0)),
            pl.BlockSpec((D, M), lambda b, c: (0, 0)),
            pl.BlockSpec((1, D), lambda b, c: (0, 0)),
        ],
        out_specs=[
            pl.BlockSpec((1, C, D), lambda b, c: (b, c, 0)),
            pl.BlockSpec((1, M, M), lambda b, c: (b, 0, 0)),
        ],
        out_shape=[
            jax.ShapeDtypeStruct((B, S, D), x.dtype),
            jax.ShapeDtypeStruct((B, M, M), x.dtype),
        ],
        compiler_params=pltpu.CompilerParams(
            dimension_semantics=("parallel", "arbitrary"),
        ),
    )(x, Wk, bk.reshape(1, M), Wv, bv.reshape(1, M), Wq, bq.reshape(1, M),
      Wo, bo.reshape(1, D))
    return (y, state)
